# baseline (speedup 1.0000x reference)
import functools

import jax
import jax.numpy as jnp
from jax import lax
from jax.experimental import pallas as pl
from jax.experimental.pallas import tpu as pltpu
from jax.experimental.pallas import tpu_sc as plsc


def _dot(a, b):
    return lax.dot_general(
        a, b, (((1,), (0,)), ((), ())),
        precision=lax.Precision.HIGHEST,
        preferred_element_type=jnp.float32,
    )


def _h_matmul_kernel(x_ref, w_ref, h_ref):
    hfull = _dot(x_ref[...], w_ref[0])
    nb = hfull.shape[0]
    for q in range(4):
        a = hfull[:, q * 32:q * 32 + 16]
        b = hfull[:, q * 32 + 16:q * 32 + 32]
        inter = jnp.stack([a, b], axis=-1).reshape(nb, 32)
        h_ref[q, 0] = inter.astype(jnp.bfloat16)


def _out_kernel(a_ref, x_ref, root_ref, bias_ref, o_ref):
    agg = jnp.concatenate(
        [a_ref[0], a_ref[1], a_ref[2], a_ref[3]], axis=1)
    acc = agg + _dot(x_ref[...], root_ref[...])
    o_ref[...] = jnp.maximum(acc + bias_ref[...], 0.0)


def _valid_mask(base, limit):
    idx = base + lax.iota(jnp.int32, 16)
    return idx < limit


def _make_agg_kernel(n, r, h_dim, e, mesh):
    nr = n * r
    hq = h_dim // 4
    epw = e // 16
    g = (epw + 127) // 128
    epad = g * 128
    per_tile_c = nr // 16
    cpad = ((per_tile_c + 15) // 16) * 16
    nfull_blk = n // 128
    tail_rows = n - nfull_blk * 128
    nblk = nfull_blk + (1 if tail_rows else 0)
    blk_iters = (nblk + 15) // 16

    @functools.partial(
        pl.kernel,
        out_type=jax.ShapeDtypeStruct((4, n, hq), jnp.float32),
        mesh=mesh,
        compiler_params=pltpu.CompilerParams(use_tc_tiling_on_sc=False),
        scratch_types=[
            pltpu.VMEM((epad,), jnp.int32),
            pltpu.VMEM((epad,), jnp.int32),
            pltpu.VMEM((epad,), jnp.int32),
            pltpu.VMEM((1, 128), jnp.int32),
            pltpu.VMEM((1, 128), jnp.int32),
            pltpu.VMEM((1, 128), jnp.int32),
            pltpu.VMEM((1, 128), jnp.int32),
            pltpu.VMEM((1, 128), jnp.int32),
            pltpu.VMEM((1, 128), jnp.int32),
            pltpu.VMEM((1, 128), jnp.float32),
            pltpu.VMEM((1, 128), jnp.float32),
            pltpu.VMEM((128, hq // 2), jnp.int32),
            pltpu.VMEM((128, hq // 2), jnp.int32),
            pltpu.VMEM((128, hq), jnp.float32),
            pltpu.VMEM((128, hq), jnp.float32),
            pltpu.VMEM((cpad,), jnp.float32),
            pltpu.SemaphoreType.DMA,
            pltpu.SemaphoreType.DMA,
            pltpu.SemaphoreType.DMA,
            pltpu.SemaphoreType.DMA,
            pltpu.VMEM_SHARED((nr,), jnp.float32),
            pltpu.VMEM_SHARED((n, hq), jnp.float32),
        ],
    )
    def agg_kernel(h0_hbm, h1_hbm, h2_hbm, h3_hbm,
                   src_hbm, dst_hbm, typ_hbm, out_hbm,
                   src_v, dst_v, typ_v, row0_v, row1_v, dst0_v, dst1_v,
                   key0_v, key1_v, nrm0_v, nrm1_v, hg0_v, hg1_v, hs0_v, hs1_v,
                   cz_v,
                   gsem0, gsem1, ssem0, ssem1,
                   inv_sh, agg_sh):
        h_q = (h0_hbm, h1_hbm, h2_hbm, h3_hbm)
        row_s = (row0_v, row1_v)
        dstix_s = (dst0_v, dst1_v)
        key_s = (key0_v, key1_v)
        norm_s = (nrm0_v, nrm1_v)
        hg_s = (hg0_v, hg1_v)
        hs_s = (hs0_v, hs1_v)
        gsem = (gsem0, gsem1)
        ssem = (ssem0, ssem1)
        cid = lax.axis_index("c")
        sid = lax.axis_index("s")
        base = sid * epw

        pltpu.sync_copy(src_hbm.at[pl.ds(base, epw)], src_v.at[pl.ds(0, epw)])
        pltpu.sync_copy(dst_hbm.at[pl.ds(base, epw)], dst_v.at[pl.ds(0, epw)])
        pltpu.sync_copy(typ_hbm.at[pl.ds(base, epw)], typ_v.at[pl.ds(0, epw)])

        cbase = pl.multiple_of(sid * per_tile_c, 8)

        def czero(i, c):
            cz_v[pl.ds(pl.multiple_of(i * 16, 16), 16)] = jnp.zeros(
                (16,), jnp.float32)
            return c
        lax.fori_loop(0, cpad // 16, czero, 0)
        pltpu.sync_copy(cz_v.at[pl.ds(0, per_tile_c)],
                        inv_sh.at[pl.ds(cbase, per_tile_c)])
        plsc.subcore_barrier()

        def cntbody(j, c):
            for v in range(8):
                off = pl.multiple_of(j * 128 + v * 16, 16)
                valid = _valid_mask(off, epw)
                d = dst_v[pl.ds(off, 16)]
                t = typ_v[pl.ds(off, 16)]
                key0_v[0, pl.ds(v * 16, 16)] = jnp.where(valid, d * r + t, 0)
                nrm0_v[0, pl.ds(v * 16, 16)] = jnp.where(valid, 1.0, 0.0)
            pltpu.sync_copy(nrm0_v.at[0], inv_sh.at[key0_v.at[0]], add=True)
            return c
        lax.fori_loop(0, g, cntbody, 0)
        plsc.subcore_barrier()

        pltpu.sync_copy(inv_sh.at[pl.ds(cbase, per_tile_c)],
                        cz_v.at[pl.ds(0, per_tile_c)])

        def cbody(i, c):
            off = pl.multiple_of(i * 16, 16)
            cz_v[pl.ds(off, 16)] = 1.0 / jnp.maximum(cz_v[pl.ds(off, 16)], 1.0)
            return c
        lax.fori_loop(0, cpad // 16, cbody, 0)
        pltpu.sync_copy(cz_v.at[pl.ds(0, per_tile_c)],
                        inv_sh.at[pl.ds(cbase, per_tile_c)])

        def run_pass(q):
            def zb(i, c):
                for v in range(hq // 16):
                    hs0_v[i, pl.ds(v * 16, 16)] = jnp.zeros(
                        (16,), jnp.float32)
                return c
            lax.fori_loop(0, 128, zb, 0)

            def zcopy(k, c):
                b = sid + k * 16

                @pl.when(b < nfull_blk)
                def _():
                    off = pl.multiple_of(b * 128, 128)
                    pltpu.sync_copy(hs0_v.at[pl.ds(0, 128)],
                                    agg_sh.at[pl.ds(off, 128)])
                if tail_rows:
                    @pl.when(b == nfull_blk)
                    def _():
                        pltpu.sync_copy(
                            hs0_v.at[pl.ds(0, tail_rows)],
                            agg_sh.at[pl.ds(nfull_blk * 128, tail_rows)])
                return c
            lax.fori_loop(0, blk_iters, zcopy, 0)
            plsc.subcore_barrier()

            def prefetch(j, b):
                for v in range(8):
                    off = pl.multiple_of(j * 128 + v * 16, 16)
                    valid = _valid_mask(off, epw)
                    s = src_v[pl.ds(off, 16)]
                    d = dst_v[pl.ds(off, 16)]
                    t = typ_v[pl.ds(off, 16)]
                    row_s[b][0, pl.ds(v * 16, 16)] = jnp.where(
                        valid, t * n + s, 0)
                    key_s[b][0, pl.ds(v * 16, 16)] = jnp.where(
                        valid, d * r + t, 0)

                @pl.when(cid == 0)
                def _():
                    pltpu.async_copy(
                        h_q[q].at[row_s[b].at[0]], hg_s[b], gsem[b])

                @pl.when(cid == 1)
                def _():
                    pltpu.async_copy(
                        h_q[2 + q].at[row_s[b].at[0]], hg_s[b], gsem[b])

            def process(j, b):
                pltpu.make_async_copy(
                    h_q[q].at[row_s[b].at[0]], hg_s[b], gsem[b]).wait()
                pltpu.sync_copy(inv_sh.at[key_s[b].at[0]], norm_s[b].at[0])

                @pl.when(j >= 2)
                def _():
                    pltpu.make_async_copy(
                        hs_s[b], agg_sh.at[dstix_s[b].at[0]], ssem[b]).wait()

                for v in range(8):
                    off = pl.multiple_of(j * 128 + v * 16, 16)
                    valid = _valid_mask(off, epw)
                    d = dst_v[pl.ds(off, 16)]
                    dstix_s[b][0, pl.ds(v * 16, 16)] = jnp.where(valid, d, 0)

                def srow(m, c2):
                    moff = pl.multiple_of(m * 16, 16)
                    valid = _valid_mask(j * 128 + moff, epw)
                    nv16 = jnp.where(valid, norm_s[b][0, pl.ds(moff, 16)], 0.0)
                    for i16 in range(16):
                        nb = lax.gather(
                            nv16, jnp.full((16, 1), i16, jnp.int32),
                            lax.GatherDimensionNumbers(
                                offset_dims=(), collapsed_slice_dims=(0,),
                                start_index_map=(0,)),
                            (1,),
                            mode=lax.GatherScatterMode.PROMISE_IN_BOUNDS)
                        row = m * 16 + i16
                        packed = hg_s[b][row, :]
                        lo = lax.bitcast_convert_type(
                            lax.shift_left(packed, 16), jnp.float32)
                        hi = lax.bitcast_convert_type(
                            lax.bitwise_and(packed, jnp.int32(-65536)),
                            jnp.float32)
                        hs_s[b][row, pl.ds(0, 16)] = lo * nb
                        hs_s[b][row, pl.ds(16, 16)] = hi * nb
                    return c2
                lax.fori_loop(0, 8, srow, 0)
                pltpu.async_copy(hs_s[b], agg_sh.at[dstix_s[b].at[0]],
                                 ssem[b], add=True)

                @pl.when(j + 2 < g)
                def _():
                    prefetch(j + 2, b)

            prefetch(0, 0)
            prefetch(1, 1)

            def outer(j2, c):
                for b in range(2):
                    j = j2 * 2 + b

                    @pl.when(j < g)
                    def _():
                        process(j, b)
                return c
            lax.fori_loop(0, (g + 1) // 2, outer, 0)

            pltpu.make_async_copy(
                hs1_v, agg_sh.at[dst1_v.at[0]], ssem[1]).wait()
            pltpu.make_async_copy(
                hs0_v, agg_sh.at[dst0_v.at[0]], ssem[0]).wait()
            plsc.subcore_barrier()

            oq = cid * 2 + q

            def wcopy(k, c):
                b = sid + k * 16

                @pl.when(b < nfull_blk)
                def _():
                    off = pl.multiple_of(b * 128, 128)
                    pltpu.sync_copy(agg_sh.at[pl.ds(off, 128)], hs0_v)
                    pltpu.sync_copy(hs0_v, out_hbm.at[oq, pl.ds(off, 128)])
                if tail_rows:
                    @pl.when(b == nfull_blk)
                    def _():
                        pltpu.sync_copy(
                            agg_sh.at[pl.ds(nfull_blk * 128, tail_rows)],
                            hs0_v.at[pl.ds(0, tail_rows)])
                        pltpu.sync_copy(
                            hs0_v.at[pl.ds(0, tail_rows)],
                            out_hbm.at[oq, pl.ds(nfull_blk * 128, tail_rows)])
                return c
            lax.fori_loop(0, blk_iters, wcopy, 0)
            plsc.subcore_barrier()

        run_pass(0)
        run_pass(1)

    return agg_kernel


def kernel(x, edge_index, edge_type, W, root, bias):
    n, d = x.shape
    r, _, h_dim = W.shape
    e = edge_type.shape[0]
    hq = h_dim // 4

    src = edge_index[0]
    dst = edge_index[1]

    nb = 2000
    h = pl.pallas_call(
        _h_matmul_kernel,
        grid=(n // nb, r),
        in_specs=[
            pl.BlockSpec((nb, d), lambda i, j: (i, 0)),
            pl.BlockSpec((1, d, h_dim), lambda i, j: (j, 0, 0)),
        ],
        out_specs=pl.BlockSpec((4, 1, nb, hq), lambda i, j: (0, j, i, 0)),
        out_shape=jax.ShapeDtypeStruct((4, r, n, hq), jnp.bfloat16),
    )(x, W)
    h32 = lax.bitcast_convert_type(
        h.reshape(4, r * n, hq // 2, 2), jnp.int32)
    hqs = [h32[q] for q in range(4)]

    mesh = plsc.VectorSubcoreMesh(core_axis_name="c", subcore_axis_name="s")

    agg_kernel = _make_agg_kernel(n, r, h_dim, e, mesh)
    agg = agg_kernel(hqs[0], hqs[1], hqs[2], hqs[3], src, dst, edge_type)

    nb2 = 2000
    out = pl.pallas_call(
        _out_kernel,
        grid=(n // nb2,),
        in_specs=[
            pl.BlockSpec((4, nb2, hq), lambda i: (0, i, 0)),
            pl.BlockSpec((nb2, d), lambda i: (i, 0)),
            pl.BlockSpec((d, h_dim), lambda i: (0, 0)),
            pl.BlockSpec((1, h_dim), lambda i: (0, 0)),
        ],
        out_specs=pl.BlockSpec((nb2, h_dim), lambda i: (i, 0)),
        out_shape=jax.ShapeDtypeStruct((n, h_dim), jnp.float32),
    )(agg, x, root, bias.reshape(1, h_dim))
    return out


# speedup vs baseline: 5.4990x; 5.4990x over previous
import functools

import jax
import jax.numpy as jnp
from jax import lax
from jax.experimental import pallas as pl
from jax.experimental.pallas import tpu as pltpu
from jax.experimental.pallas import tpu_sc as plsc


def _dot(a, b):
    return lax.dot_general(
        a, b, (((1,), (0,)), ((), ())),
        precision=lax.Precision.HIGHEST,
        preferred_element_type=jnp.float32,
    )


def _h_matmul_kernel(x_ref, w_ref, h_ref):
    hfull = _dot(x_ref[...], w_ref[0])
    hq = hfull.shape[-1] // 4
    for q in range(4):
        h_ref[q, 0] = hfull[:, q * hq:(q + 1) * hq]


def _out_kernel(a_ref, x_ref, root_ref, bias_ref, o_ref):
    agg = jnp.concatenate(
        [a_ref[0], a_ref[1], a_ref[2], a_ref[3]], axis=1)
    acc = agg + _dot(x_ref[...], root_ref[...])
    o_ref[...] = jnp.maximum(acc + bias_ref[...], 0.0)


def _valid_mask(base, limit):
    idx = base + lax.iota(jnp.int32, 16)
    return idx < limit


def _make_agg_kernel(n, r, h_dim, e, mesh):
    nr = n * r
    hq = h_dim // 4
    epw = e // 16
    g = (epw + 127) // 128
    epad = g * 128
    per_tile_c = nr // 16
    cpad = ((per_tile_c + 15) // 16) * 16
    nfull_blk = n // 128
    tail_rows = n - nfull_blk * 128
    nblk = nfull_blk + (1 if tail_rows else 0)
    blk_iters = (nblk + 15) // 16

    @functools.partial(
        pl.kernel,
        out_type=jax.ShapeDtypeStruct((4, n, hq), jnp.float32),
        mesh=mesh,
        compiler_params=pltpu.CompilerParams(use_tc_tiling_on_sc=False),
        scratch_types=[
            pltpu.VMEM((epad,), jnp.int32),
            pltpu.VMEM((epad,), jnp.int32),
            pltpu.VMEM((epad,), jnp.int32),
            pltpu.VMEM((1, 128), jnp.int32),
            pltpu.VMEM((1, 128), jnp.int32),
            pltpu.VMEM((1, 128), jnp.int32),
            pltpu.VMEM((1, 128), jnp.int32),
            pltpu.VMEM((1, 128), jnp.int32),
            pltpu.VMEM((1, 128), jnp.int32),
            pltpu.VMEM((1, 128), jnp.float32),
            pltpu.VMEM((1, 128), jnp.float32),
            pltpu.VMEM((128, hq), jnp.float32),
            pltpu.VMEM((128, hq), jnp.float32),
            pltpu.VMEM((128, hq), jnp.float32),
            pltpu.VMEM((128, hq), jnp.float32),
            pltpu.VMEM((cpad,), jnp.float32),
            pltpu.SemaphoreType.DMA,
            pltpu.SemaphoreType.DMA,
            pltpu.SemaphoreType.DMA,
            pltpu.SemaphoreType.DMA,
            pltpu.SemaphoreType.DMA,
            pltpu.SemaphoreType.DMA,
            pltpu.VMEM_SHARED((nr,), jnp.float32),
            pltpu.VMEM_SHARED((n, hq), jnp.float32),
        ],
    )
    def agg_kernel(h0_hbm, h1_hbm, h2_hbm, h3_hbm,
                   src_hbm, dst_hbm, typ_hbm, out_hbm,
                   src_v, dst_v, typ_v, row0_v, row1_v, dst0_v, dst1_v,
                   key0_v, key1_v, nrm0_v, nrm1_v, hg0_v, hg1_v, hs0_v, hs1_v,
                   cz_v,
                   gsem0, gsem1, nsem0, nsem1, ssem0, ssem1,
                   inv_sh, agg_sh):
        h_q = (h0_hbm, h1_hbm, h2_hbm, h3_hbm)
        row_s = (row0_v, row1_v)
        dstix_s = (dst0_v, dst1_v)
        key_s = (key0_v, key1_v)
        norm_s = (nrm0_v, nrm1_v)
        hg_s = (hg0_v, hg1_v)
        hs_s = (hs0_v, hs1_v)
        gsem = (gsem0, gsem1)
        nsem = (nsem0, nsem1)
        ssem = (ssem0, ssem1)
        cid = lax.axis_index("c")
        sid = lax.axis_index("s")
        base = sid * epw

        pltpu.sync_copy(src_hbm.at[pl.ds(base, epw)], src_v.at[pl.ds(0, epw)])
        pltpu.sync_copy(dst_hbm.at[pl.ds(base, epw)], dst_v.at[pl.ds(0, epw)])
        pltpu.sync_copy(typ_hbm.at[pl.ds(base, epw)], typ_v.at[pl.ds(0, epw)])

        cbase = pl.multiple_of(sid * per_tile_c, 8)

        def czero(i, c):
            cz_v[pl.ds(pl.multiple_of(i * 16, 16), 16)] = jnp.zeros(
                (16,), jnp.float32)
            return c
        lax.fori_loop(0, cpad // 16, czero, 0)
        pltpu.sync_copy(cz_v.at[pl.ds(0, per_tile_c)],
                        inv_sh.at[pl.ds(cbase, per_tile_c)])
        plsc.subcore_barrier()

        def cntbody(j, c):
            for v in range(8):
                off = pl.multiple_of(j * 128 + v * 16, 16)
                valid = _valid_mask(off, epw)
                d = dst_v[pl.ds(off, 16)]
                t = typ_v[pl.ds(off, 16)]
                key0_v[0, pl.ds(v * 16, 16)] = jnp.where(valid, d * r + t, 0)
                nrm0_v[0, pl.ds(v * 16, 16)] = jnp.where(valid, 1.0, 0.0)
            pltpu.sync_copy(nrm0_v.at[0], inv_sh.at[key0_v.at[0]], add=True)
            return c
        lax.fori_loop(0, g, cntbody, 0)
        plsc.subcore_barrier()

        pltpu.sync_copy(inv_sh.at[pl.ds(cbase, per_tile_c)],
                        cz_v.at[pl.ds(0, per_tile_c)])

        def cbody(i, c):
            off = pl.multiple_of(i * 16, 16)
            cz_v[pl.ds(off, 16)] = 1.0 / jnp.maximum(cz_v[pl.ds(off, 16)], 1.0)
            return c
        lax.fori_loop(0, cpad // 16, cbody, 0)
        pltpu.sync_copy(cz_v.at[pl.ds(0, per_tile_c)],
                        inv_sh.at[pl.ds(cbase, per_tile_c)])

        def run_pass(q):
            def zb(i, c):
                for v in range(hq // 16):
                    hs0_v[i, pl.ds(v * 16, 16)] = jnp.zeros(
                        (16,), jnp.float32)
                return c
            lax.fori_loop(0, 128, zb, 0)

            def zcopy(k, c):
                b = sid + k * 16

                @pl.when(b < nfull_blk)
                def _():
                    off = pl.multiple_of(b * 128, 128)
                    pltpu.sync_copy(hs0_v.at[pl.ds(0, 128)],
                                    agg_sh.at[pl.ds(off, 128)])
                if tail_rows:
                    @pl.when(b == nfull_blk)
                    def _():
                        pltpu.sync_copy(
                            hs0_v.at[pl.ds(0, tail_rows)],
                            agg_sh.at[pl.ds(nfull_blk * 128, tail_rows)])
                return c
            lax.fori_loop(0, blk_iters, zcopy, 0)
            plsc.subcore_barrier()

            def prefetch(j, b):
                for v in range(8):
                    off = pl.multiple_of(j * 128 + v * 16, 16)
                    valid = _valid_mask(off, epw)
                    s = src_v[pl.ds(off, 16)]
                    d = dst_v[pl.ds(off, 16)]
                    t = typ_v[pl.ds(off, 16)]
                    row_s[b][0, pl.ds(v * 16, 16)] = jnp.where(
                        valid, t * n + s, 0)
                    key_s[b][0, pl.ds(v * 16, 16)] = jnp.where(
                        valid, d * r + t, 0)
                pltpu.async_copy(
                    inv_sh.at[key_s[b].at[0]], norm_s[b].at[0], nsem[b])

                @pl.when(cid == 0)
                def _():
                    pltpu.async_copy(
                        h_q[q].at[row_s[b].at[0]], hg_s[b], gsem[b])

                @pl.when(cid == 1)
                def _():
                    pltpu.async_copy(
                        h_q[2 + q].at[row_s[b].at[0]], hg_s[b], gsem[b])

            def process(j, b):
                pltpu.make_async_copy(
                    h_q[q].at[row_s[b].at[0]], hg_s[b], gsem[b]).wait()
                pltpu.make_async_copy(
                    inv_sh.at[key_s[b].at[0]], norm_s[b].at[0],
                    nsem[b]).wait()

                @pl.when(j >= 2)
                def _():
                    pltpu.make_async_copy(
                        hs_s[b], agg_sh.at[dstix_s[b].at[0]], ssem[b]).wait()

                for v in range(8):
                    off = pl.multiple_of(j * 128 + v * 16, 16)
                    valid = _valid_mask(off, epw)
                    d = dst_v[pl.ds(off, 16)]
                    dstix_s[b][0, pl.ds(v * 16, 16)] = jnp.where(valid, d, 0)

                def srow(m, c2):
                    moff = pl.multiple_of(m * 16, 16)
                    valid = _valid_mask(j * 128 + moff, epw)
                    nv16 = jnp.where(valid, norm_s[b][0, pl.ds(moff, 16)], 0.0)
                    for i16 in range(16):
                        nb = lax.gather(
                            nv16, jnp.full((16, 1), i16, jnp.int32),
                            lax.GatherDimensionNumbers(
                                offset_dims=(), collapsed_slice_dims=(0,),
                                start_index_map=(0,)),
                            (1,),
                            mode=lax.GatherScatterMode.PROMISE_IN_BOUNDS)
                        row = m * 16 + i16
                        for v in range(hq // 16):
                            hs_s[b][row, pl.ds(v * 16, 16)] = (
                                hg_s[b][row, pl.ds(v * 16, 16)] * nb)
                    return c2
                lax.fori_loop(0, 8, srow, 0)
                pltpu.async_copy(hs_s[b], agg_sh.at[dstix_s[b].at[0]],
                                 ssem[b], add=True)

                @pl.when(j + 2 < g)
                def _():
                    prefetch(j + 2, b)

            prefetch(0, 0)
            prefetch(1, 1)

            def outer(j2, c):
                for b in range(2):
                    j = j2 * 2 + b

                    @pl.when(j < g)
                    def _():
                        process(j, b)
                return c
            lax.fori_loop(0, (g + 1) // 2, outer, 0)

            pltpu.make_async_copy(
                hs1_v, agg_sh.at[dst1_v.at[0]], ssem[1]).wait()
            pltpu.make_async_copy(
                hs0_v, agg_sh.at[dst0_v.at[0]], ssem[0]).wait()
            plsc.subcore_barrier()

            oq = cid * 2 + q

            def wcopy(k, c):
                b = sid + k * 16

                @pl.when(b < nfull_blk)
                def _():
                    off = pl.multiple_of(b * 128, 128)
                    pltpu.sync_copy(agg_sh.at[pl.ds(off, 128)], hs0_v)
                    pltpu.sync_copy(hs0_v, out_hbm.at[oq, pl.ds(off, 128)])
                if tail_rows:
                    @pl.when(b == nfull_blk)
                    def _():
                        pltpu.sync_copy(
                            agg_sh.at[pl.ds(nfull_blk * 128, tail_rows)],
                            hs0_v.at[pl.ds(0, tail_rows)])
                        pltpu.sync_copy(
                            hs0_v.at[pl.ds(0, tail_rows)],
                            out_hbm.at[oq, pl.ds(nfull_blk * 128, tail_rows)])
                return c
            lax.fori_loop(0, blk_iters, wcopy, 0)
            plsc.subcore_barrier()

        run_pass(0)
        run_pass(1)

    return agg_kernel


def kernel(x, edge_index, edge_type, W, root, bias):
    n, d = x.shape
    r, _, h_dim = W.shape
    e = edge_type.shape[0]
    hq = h_dim // 4

    src = edge_index[0]
    dst = edge_index[1]

    nb = 2000
    h = pl.pallas_call(
        _h_matmul_kernel,
        grid=(n // nb, r),
        in_specs=[
            pl.BlockSpec((nb, d), lambda i, j: (i, 0)),
            pl.BlockSpec((1, d, h_dim), lambda i, j: (j, 0, 0)),
        ],
        out_specs=pl.BlockSpec((4, 1, nb, hq), lambda i, j: (0, j, i, 0)),
        out_shape=jax.ShapeDtypeStruct((4, r, n, hq), jnp.float32),
    )(x, W)
    hqs = [h[q].reshape(r * n, hq) for q in range(4)]

    mesh = plsc.VectorSubcoreMesh(core_axis_name="c", subcore_axis_name="s")

    agg_kernel = _make_agg_kernel(n, r, h_dim, e, mesh)
    agg = agg_kernel(hqs[0], hqs[1], hqs[2], hqs[3], src, dst, edge_type)

    nb2 = 2000
    out = pl.pallas_call(
        _out_kernel,
        grid=(n // nb2,),
        in_specs=[
            pl.BlockSpec((4, nb2, hq), lambda i: (0, i, 0)),
            pl.BlockSpec((nb2, d), lambda i: (i, 0)),
            pl.BlockSpec((d, h_dim), lambda i: (0, 0)),
            pl.BlockSpec((1, h_dim), lambda i: (0, 0)),
        ],
        out_specs=pl.BlockSpec((nb2, h_dim), lambda i: (i, 0)),
        out_shape=jax.ShapeDtypeStruct((n, h_dim), jnp.float32),
    )(agg, x, root, bias.reshape(1, h_dim))
    return out


# speedup vs baseline: 5.5857x; 1.0158x over previous
import functools

import jax
import jax.numpy as jnp
from jax import lax
from jax.experimental import pallas as pl
from jax.experimental.pallas import tpu as pltpu
from jax.experimental.pallas import tpu_sc as plsc


def _dot(a, b):
    return lax.dot_general(
        a, b, (((1,), (0,)), ((), ())),
        preferred_element_type=jnp.float32,
    )


def _h_matmul_kernel(x_ref, w_ref, h_ref):
    hfull = _dot(x_ref[...], w_ref[0])
    hq = hfull.shape[-1] // 4
    for q in range(4):
        h_ref[q, 0] = hfull[:, q * hq:(q + 1) * hq]


def _out_kernel(a_ref, x_ref, root_ref, bias_ref, o_ref):
    agg = jnp.concatenate(
        [a_ref[0], a_ref[1], a_ref[2], a_ref[3]], axis=1)
    acc = agg + _dot(x_ref[...], root_ref[...])
    o_ref[...] = jnp.maximum(acc + bias_ref[...], 0.0)


def _valid_mask(base, limit):
    idx = base + lax.iota(jnp.int32, 16)
    return idx < limit


def _make_agg_kernel(n, r, h_dim, e, mesh):
    nr = n * r
    hq = h_dim // 4
    epw = e // 16
    g = (epw + 127) // 128
    epad = g * 128
    per_tile_c = nr // 16
    cpad = ((per_tile_c + 15) // 16) * 16
    nfull_blk = n // 128
    tail_rows = n - nfull_blk * 128
    nblk = nfull_blk + (1 if tail_rows else 0)
    blk_iters = (nblk + 15) // 16

    @functools.partial(
        pl.kernel,
        out_type=jax.ShapeDtypeStruct((4, n, hq), jnp.float32),
        mesh=mesh,
        compiler_params=pltpu.CompilerParams(use_tc_tiling_on_sc=False),
        scratch_types=[
            pltpu.VMEM((epad,), jnp.int32),
            pltpu.VMEM((epad,), jnp.int32),
            pltpu.VMEM((epad,), jnp.int32),
            pltpu.VMEM((1, 128), jnp.int32),
            pltpu.VMEM((1, 128), jnp.int32),
            pltpu.VMEM((1, 128), jnp.int32),
            pltpu.VMEM((1, 128), jnp.int32),
            pltpu.VMEM((1, 128), jnp.int32),
            pltpu.VMEM((1, 128), jnp.int32),
            pltpu.VMEM((1, 128), jnp.float32),
            pltpu.VMEM((1, 128), jnp.float32),
            pltpu.VMEM((128, hq), jnp.float32),
            pltpu.VMEM((128, hq), jnp.float32),
            pltpu.VMEM((128, hq), jnp.float32),
            pltpu.VMEM((128, hq), jnp.float32),
            pltpu.VMEM((cpad,), jnp.float32),
            pltpu.SemaphoreType.DMA,
            pltpu.SemaphoreType.DMA,
            pltpu.SemaphoreType.DMA,
            pltpu.SemaphoreType.DMA,
            pltpu.SemaphoreType.DMA,
            pltpu.SemaphoreType.DMA,
            pltpu.VMEM_SHARED((nr,), jnp.float32),
            pltpu.VMEM_SHARED((n, hq), jnp.float32),
        ],
    )
    def agg_kernel(h0_hbm, h1_hbm, h2_hbm, h3_hbm,
                   src_hbm, dst_hbm, typ_hbm, out_hbm,
                   src_v, dst_v, typ_v, row0_v, row1_v, dst0_v, dst1_v,
                   key0_v, key1_v, nrm0_v, nrm1_v, hg0_v, hg1_v, hs0_v, hs1_v,
                   cz_v,
                   gsem0, gsem1, nsem0, nsem1, ssem0, ssem1,
                   inv_sh, agg_sh):
        h_q = (h0_hbm, h1_hbm, h2_hbm, h3_hbm)
        row_s = (row0_v, row1_v)
        dstix_s = (dst0_v, dst1_v)
        key_s = (key0_v, key1_v)
        norm_s = (nrm0_v, nrm1_v)
        hg_s = (hg0_v, hg1_v)
        hs_s = (hs0_v, hs1_v)
        gsem = (gsem0, gsem1)
        nsem = (nsem0, nsem1)
        ssem = (ssem0, ssem1)
        cid = lax.axis_index("c")
        sid = lax.axis_index("s")
        base = sid * epw

        pltpu.sync_copy(src_hbm.at[pl.ds(base, epw)], src_v.at[pl.ds(0, epw)])
        pltpu.sync_copy(dst_hbm.at[pl.ds(base, epw)], dst_v.at[pl.ds(0, epw)])
        pltpu.sync_copy(typ_hbm.at[pl.ds(base, epw)], typ_v.at[pl.ds(0, epw)])

        cbase = pl.multiple_of(sid * per_tile_c, 8)

        def czero(i, c):
            cz_v[pl.ds(pl.multiple_of(i * 16, 16), 16)] = jnp.zeros(
                (16,), jnp.float32)
            return c
        lax.fori_loop(0, cpad // 16, czero, 0)
        pltpu.sync_copy(cz_v.at[pl.ds(0, per_tile_c)],
                        inv_sh.at[pl.ds(cbase, per_tile_c)])
        plsc.subcore_barrier()

        def cntbody(j, c):
            for v in range(8):
                off = pl.multiple_of(j * 128 + v * 16, 16)
                valid = _valid_mask(off, epw)
                d = dst_v[pl.ds(off, 16)]
                t = typ_v[pl.ds(off, 16)]
                key0_v[0, pl.ds(v * 16, 16)] = jnp.where(valid, d * r + t, 0)
                nrm0_v[0, pl.ds(v * 16, 16)] = jnp.where(valid, 1.0, 0.0)
            pltpu.sync_copy(nrm0_v.at[0], inv_sh.at[key0_v.at[0]], add=True)
            return c
        lax.fori_loop(0, g, cntbody, 0)
        plsc.subcore_barrier()

        pltpu.sync_copy(inv_sh.at[pl.ds(cbase, per_tile_c)],
                        cz_v.at[pl.ds(0, per_tile_c)])

        def cbody(i, c):
            off = pl.multiple_of(i * 16, 16)
            cz_v[pl.ds(off, 16)] = 1.0 / jnp.maximum(cz_v[pl.ds(off, 16)], 1.0)
            return c
        lax.fori_loop(0, cpad // 16, cbody, 0)
        pltpu.sync_copy(cz_v.at[pl.ds(0, per_tile_c)],
                        inv_sh.at[pl.ds(cbase, per_tile_c)])

        def run_pass(q):
            def zb(i, c):
                for v in range(hq // 16):
                    hs0_v[i, pl.ds(v * 16, 16)] = jnp.zeros(
                        (16,), jnp.float32)
                return c
            lax.fori_loop(0, 128, zb, 0)

            def zcopy(k, c):
                b = sid + k * 16

                @pl.when(b < nfull_blk)
                def _():
                    off = pl.multiple_of(b * 128, 128)
                    pltpu.sync_copy(hs0_v.at[pl.ds(0, 128)],
                                    agg_sh.at[pl.ds(off, 128)])
                if tail_rows:
                    @pl.when(b == nfull_blk)
                    def _():
                        pltpu.sync_copy(
                            hs0_v.at[pl.ds(0, tail_rows)],
                            agg_sh.at[pl.ds(nfull_blk * 128, tail_rows)])
                return c
            lax.fori_loop(0, blk_iters, zcopy, 0)
            plsc.subcore_barrier()

            def prefetch(j, b):
                for v in range(8):
                    off = pl.multiple_of(j * 128 + v * 16, 16)
                    valid = _valid_mask(off, epw)
                    s = src_v[pl.ds(off, 16)]
                    d = dst_v[pl.ds(off, 16)]
                    t = typ_v[pl.ds(off, 16)]
                    row_s[b][0, pl.ds(v * 16, 16)] = jnp.where(
                        valid, t * n + s, 0)
                    key_s[b][0, pl.ds(v * 16, 16)] = jnp.where(
                        valid, d * r + t, 0)
                pltpu.async_copy(
                    inv_sh.at[key_s[b].at[0]], norm_s[b].at[0], nsem[b])

                @pl.when(cid == 0)
                def _():
                    pltpu.async_copy(
                        h_q[q].at[row_s[b].at[0]], hg_s[b], gsem[b])

                @pl.when(cid == 1)
                def _():
                    pltpu.async_copy(
                        h_q[2 + q].at[row_s[b].at[0]], hg_s[b], gsem[b])

            def process(j, b):
                pltpu.make_async_copy(
                    h_q[q].at[row_s[b].at[0]], hg_s[b], gsem[b]).wait()
                pltpu.make_async_copy(
                    inv_sh.at[key_s[b].at[0]], norm_s[b].at[0],
                    nsem[b]).wait()

                @pl.when(j >= 2)
                def _():
                    pltpu.make_async_copy(
                        hs_s[b], agg_sh.at[dstix_s[b].at[0]], ssem[b]).wait()

                for v in range(8):
                    off = pl.multiple_of(j * 128 + v * 16, 16)
                    valid = _valid_mask(off, epw)
                    d = dst_v[pl.ds(off, 16)]
                    dstix_s[b][0, pl.ds(v * 16, 16)] = jnp.where(valid, d, 0)

                def srow(m, c2):
                    moff = pl.multiple_of(m * 16, 16)
                    valid = _valid_mask(j * 128 + moff, epw)
                    nv16 = jnp.where(valid, norm_s[b][0, pl.ds(moff, 16)], 0.0)
                    for i16 in range(16):
                        nb = lax.gather(
                            nv16, jnp.full((16, 1), i16, jnp.int32),
                            lax.GatherDimensionNumbers(
                                offset_dims=(), collapsed_slice_dims=(0,),
                                start_index_map=(0,)),
                            (1,),
                            mode=lax.GatherScatterMode.PROMISE_IN_BOUNDS)
                        row = m * 16 + i16
                        for v in range(hq // 16):
                            hs_s[b][row, pl.ds(v * 16, 16)] = (
                                hg_s[b][row, pl.ds(v * 16, 16)] * nb)
                    return c2
                lax.fori_loop(0, 8, srow, 0)
                pltpu.async_copy(hs_s[b], agg_sh.at[dstix_s[b].at[0]],
                                 ssem[b], add=True)

                @pl.when(j + 2 < g)
                def _():
                    prefetch(j + 2, b)

            prefetch(0, 0)
            prefetch(1, 1)

            def outer(j2, c):
                for b in range(2):
                    j = j2 * 2 + b

                    @pl.when(j < g)
                    def _():
                        process(j, b)
                return c
            lax.fori_loop(0, (g + 1) // 2, outer, 0)

            pltpu.make_async_copy(
                hs1_v, agg_sh.at[dst1_v.at[0]], ssem[1]).wait()
            pltpu.make_async_copy(
                hs0_v, agg_sh.at[dst0_v.at[0]], ssem[0]).wait()
            plsc.subcore_barrier()

            oq = cid * 2 + q

            def wcopy(k, c):
                b = sid + k * 16

                @pl.when(b < nfull_blk)
                def _():
                    off = pl.multiple_of(b * 128, 128)
                    pltpu.sync_copy(agg_sh.at[pl.ds(off, 128)], hs0_v)
                    pltpu.sync_copy(hs0_v, out_hbm.at[oq, pl.ds(off, 128)])
                if tail_rows:
                    @pl.when(b == nfull_blk)
                    def _():
                        pltpu.sync_copy(
                            agg_sh.at[pl.ds(nfull_blk * 128, tail_rows)],
                            hs0_v.at[pl.ds(0, tail_rows)])
                        pltpu.sync_copy(
                            hs0_v.at[pl.ds(0, tail_rows)],
                            out_hbm.at[oq, pl.ds(nfull_blk * 128, tail_rows)])
                return c
            lax.fori_loop(0, blk_iters, wcopy, 0)
            plsc.subcore_barrier()

        run_pass(0)
        run_pass(1)

    return agg_kernel


def kernel(x, edge_index, edge_type, W, root, bias):
    n, d = x.shape
    r, _, h_dim = W.shape
    e = edge_type.shape[0]
    hq = h_dim // 4

    src = edge_index[0]
    dst = edge_index[1]

    nb = 2000
    h = pl.pallas_call(
        _h_matmul_kernel,
        grid=(n // nb, r),
        in_specs=[
            pl.BlockSpec((nb, d), lambda i, j: (i, 0)),
            pl.BlockSpec((1, d, h_dim), lambda i, j: (j, 0, 0)),
        ],
        out_specs=pl.BlockSpec((4, 1, nb, hq), lambda i, j: (0, j, i, 0)),
        out_shape=jax.ShapeDtypeStruct((4, r, n, hq), jnp.float32),
    )(x, W)
    hqs = [h[q].reshape(r * n, hq) for q in range(4)]

    mesh = plsc.VectorSubcoreMesh(core_axis_name="c", subcore_axis_name="s")

    agg_kernel = _make_agg_kernel(n, r, h_dim, e, mesh)
    agg = agg_kernel(hqs[0], hqs[1], hqs[2], hqs[3], src, dst, edge_type)

    nb2 = 2000
    out = pl.pallas_call(
        _out_kernel,
        grid=(n // nb2,),
        in_specs=[
            pl.BlockSpec((4, nb2, hq), lambda i: (0, i, 0)),
            pl.BlockSpec((nb2, d), lambda i: (i, 0)),
            pl.BlockSpec((d, h_dim), lambda i: (0, 0)),
            pl.BlockSpec((1, h_dim), lambda i: (0, 0)),
        ],
        out_specs=pl.BlockSpec((nb2, h_dim), lambda i: (i, 0)),
        out_shape=jax.ShapeDtypeStruct((n, h_dim), jnp.float32),
    )(agg, x, root, bias.reshape(1, h_dim))
    return out
